# baseline (speedup 1.0000x reference)
import jax
import jax.numpy as jnp
from jax import lax
from jax.experimental import pallas as pl
from jax.experimental.pallas import tpu as pltpu
import jax.experimental.pallas.tpu_sc as plsc

C = 8192
N = 32
NC = 2
NS = 16


def _sc_gather_body(w_hbm, xrep_hbm, out_hbm, idx_v, row_v, sem):
    wid = lax.axis_index("s") * NC + lax.axis_index("c")
    pltpu.sync_copy(xrep_hbm.at[wid], idx_v)
    pltpu.async_copy(w_hbm.at[idx_v.at[pl.ds(0, 1)]], row_v, sem).wait()
    pltpu.sync_copy(row_v, out_hbm.at[pl.ds(wid, 1)])


_sc_gather = pl.kernel(
    _sc_gather_body,
    out_type=jax.ShapeDtypeStruct((N, C), jnp.float32),
    mesh=plsc.VectorSubcoreMesh(core_axis_name="c", subcore_axis_name="s"),
    scratch_types=[
        pltpu.VMEM((8,), jnp.int32),
        pltpu.VMEM((1, C), jnp.float32),
        pltpu.SemaphoreType.DMA,
    ],
)


def _tc_ce_body(logits_ref, y_ref, loss_ref):
    l = logits_ref[...]
    m = jnp.max(l, axis=1, keepdims=True)
    s = jnp.sum(jnp.exp(l - m), axis=1, keepdims=True)
    cols = lax.broadcasted_iota(jnp.int32, l.shape, 1)
    t = jnp.sum(jnp.where(cols == y_ref[...], l, 0.0), axis=1, keepdims=True)
    nll = jnp.log(s) + m - t
    loss_ref[...] = jnp.sum(nll, axis=0, keepdims=True) / N


def kernel(x, y, W):
    xf = x.reshape(N).astype(jnp.int32)
    xrep = jnp.broadcast_to(xf[:, None], (N, 8))
    logits = _sc_gather(W, xrep)
    y2 = y.reshape(N, 1).astype(jnp.int32)
    loss = pl.pallas_call(
        _tc_ce_body,
        out_shape=jax.ShapeDtypeStruct((1, 1), jnp.float32),
    )(logits, y2)
    return logits, loss[0, 0]
